# Initial kernel scaffold; baseline (speedup 1.0000x reference)
import jax
import jax.numpy as jnp
from jax.experimental import pallas as pl


def kernel(x, edge_index, W, b):
    raise NotImplementedError("write your pallas kernel here")


# speedup vs baseline: 6.0120x; 6.0120x over previous
import functools

import jax
import jax.numpy as jnp
from jax import lax
from jax.experimental import pallas as pl
from jax.experimental.pallas import tpu as pltpu
from jax.experimental.pallas import tpu_sc as plsc

N_NODES = 10000
N_EDGES = 160000
F_IN = 256
F_OUT = 256
FH = 128
NC = 2
NS = 16

DEG_CHUNK = 125
DEG_ROWS_PER_W = (N_EDGES // (NC * NS)) // DEG_CHUNK

AGG_CHUNK = 125
AGG_ROWS_PER_W = (N_EDGES // NS) // AGG_CHUNK

STRIPE = 624
TAIL_ROWS = N_NODES - NS * STRIPE

_MESH = plsc.VectorSubcoreMesh(core_axis_name="c", subcore_axis_name="s")


@functools.partial(
    pl.kernel,
    out_type=[jax.ShapeDtypeStruct((N_NODES,), jnp.float32),
              jax.ShapeDtypeStruct((N_NODES,), jnp.float32)],
    mesh=_MESH,
    scratch_types=[
        pltpu.VMEM((DEG_ROWS_PER_W, DEG_CHUNK), jnp.int32),
        pltpu.VMEM((DEG_CHUNK,), jnp.float32),
        pltpu.VMEM_SHARED((N_NODES,), jnp.float32),
        pltpu.SemaphoreType.DMA,
    ],
)
def _deg_kernel(src_hbm, zeros_hbm, ones_hbm, deg0_hbm, deg1_hbm,
                idx_v, ones_v, deg_sh, sem):
    c = lax.axis_index("c")
    s = lax.axis_index("s")
    w = c * NS + s

    @pl.when(s == 0)
    def _():
        pltpu.sync_copy(zeros_hbm, deg_sh)

    pltpu.sync_copy(src_hbm.at[pl.ds(w * DEG_ROWS_PER_W, DEG_ROWS_PER_W)],
                    idx_v)
    pltpu.sync_copy(ones_hbm, ones_v)
    plsc.subcore_barrier()

    @pl.loop(0, DEG_ROWS_PER_W)
    def _(j):
        pltpu.sync_copy(ones_v, deg_sh.at[idx_v.at[j]], add=True)

    plsc.subcore_barrier()

    @pl.when((s == 0) & (c == 0))
    def _():
        pltpu.sync_copy(deg_sh, deg0_hbm)

    @pl.when((s == 0) & (c == 1))
    def _():
        pltpu.sync_copy(deg_sh, deg1_hbm)


_TC_BLOCK = 2000


def _tc_body(x_ref, w_ref, deg0_ref, deg1_ref, h_ref):
    deg = deg0_ref[:, 0] + deg1_ref[:, 0] + 1.0
    norm = lax.rsqrt(deg)
    h = jnp.dot(x_ref[...], w_ref[...], preferred_element_type=jnp.float32)
    h = h * norm[:, None]
    h_ref[0] = h[:, :FH]
    h_ref[1] = h[:, FH:]


def _tc_matmul(x, W, deg0, deg1):
    grid = (N_NODES // _TC_BLOCK,)
    return pl.pallas_call(
        _tc_body,
        grid=grid,
        in_specs=[
            pl.BlockSpec((_TC_BLOCK, F_IN), lambda i: (i, 0)),
            pl.BlockSpec((F_IN, F_OUT), lambda i: (0, 0)),
            pl.BlockSpec((_TC_BLOCK, 1), lambda i: (i, 0)),
            pl.BlockSpec((_TC_BLOCK, 1), lambda i: (i, 0)),
        ],
        out_specs=pl.BlockSpec((NC, _TC_BLOCK, FH), lambda i: (0, i, 0)),
        out_shape=jax.ShapeDtypeStruct((NC, N_NODES, FH), jnp.float32),
    )(x, W, deg0, deg1)


@functools.partial(
    pl.kernel,
    out_type=jax.ShapeDtypeStruct((N_NODES, F_OUT), jnp.float32),
    mesh=_MESH,
    scratch_types=[
        pltpu.VMEM((AGG_ROWS_PER_W, AGG_CHUNK), jnp.int32),
        pltpu.VMEM((AGG_ROWS_PER_W, AGG_CHUNK), jnp.int32),
        pltpu.VMEM((AGG_CHUNK, FH), jnp.float32),
        pltpu.VMEM_SHARED((N_NODES, FH), jnp.float32),
        pltpu.SemaphoreType.DMA,
    ],
)
def _agg_kernel(hcat_hbm, srcs_hbm, dst_hbm, zeros_hbm, out_hbm,
                sidx_v, didx_v, rows_v, acc_sh, sem):
    c = lax.axis_index("c")
    s = lax.axis_index("s")

    pltpu.sync_copy(zeros_hbm.at[pl.ds(s * STRIPE, STRIPE)],
                    acc_sh.at[pl.ds(s * STRIPE, STRIPE)])

    @pl.when(s == NS - 1)
    def _():
        pltpu.sync_copy(zeros_hbm.at[pl.ds(NS * STRIPE, TAIL_ROWS)],
                        acc_sh.at[pl.ds(NS * STRIPE, TAIL_ROWS)])

    pltpu.sync_copy(
        srcs_hbm.at[c].at[pl.ds(s * AGG_ROWS_PER_W, AGG_ROWS_PER_W)],
        sidx_v)
    pltpu.sync_copy(
        dst_hbm.at[pl.ds(s * AGG_ROWS_PER_W, AGG_ROWS_PER_W)],
        didx_v)
    plsc.subcore_barrier()

    @pl.loop(0, AGG_ROWS_PER_W)
    def _(j):
        pltpu.async_copy(hcat_hbm.at[sidx_v.at[j]], rows_v, sem).wait()
        pltpu.sync_copy(rows_v, acc_sh.at[didx_v.at[j]], add=True)

    plsc.subcore_barrier()

    pltpu.sync_copy(
        acc_sh.at[pl.ds(s * STRIPE, STRIPE)],
        out_hbm.at[pl.ds(s * STRIPE, STRIPE), pl.ds(c * FH, FH)])

    @pl.when(s == NS - 1)
    def _():
        pltpu.sync_copy(
            acc_sh.at[pl.ds(NS * STRIPE, TAIL_ROWS)],
            out_hbm.at[pl.ds(NS * STRIPE, TAIL_ROWS), pl.ds(c * FH, FH)])


def kernel(x, edge_index, W, b):
    src = edge_index[0].astype(jnp.int32)
    dst = edge_index[1].astype(jnp.int32)

    src_deg = src.reshape(NC * NS * DEG_ROWS_PER_W, DEG_CHUNK)
    zeros_1d = jnp.zeros((N_NODES,), jnp.float32)
    ones_c = jnp.ones((DEG_CHUNK,), jnp.float32)
    deg0, deg1 = _deg_kernel(src_deg, zeros_1d, ones_c)

    h = _tc_matmul(x, W, deg0.reshape(N_NODES, 1), deg1.reshape(N_NODES, 1))
    hcat = h.reshape(NC * N_NODES, FH)

    srcr = src.reshape(NS * AGG_ROWS_PER_W, AGG_CHUNK)
    srcs = jnp.stack([srcr, srcr + N_NODES])
    dstr = dst.reshape(NS * AGG_ROWS_PER_W, AGG_CHUNK)
    zeros_2d = jnp.zeros((N_NODES, FH), jnp.float32)

    out = _agg_kernel(hcat, srcs, dstr, zeros_2d)
    return out + b


# speedup vs baseline: 8.1454x; 1.3549x over previous
import functools

import jax
import jax.numpy as jnp
from jax import lax
from jax.experimental import pallas as pl
from jax.experimental.pallas import tpu as pltpu
from jax.experimental.pallas import tpu_sc as plsc

N_NODES = 10000
N_EDGES = 160000
F_IN = 256
F_OUT = 256
FH = 128
NC = 2
NS = 16

DEG_CHUNK = 125
DEG_ROWS_PER_W = (N_EDGES // (NC * NS)) // DEG_CHUNK

AGG_CHUNK = 125
AGG_ROWS_PER_W = (N_EDGES // NS) // AGG_CHUNK

STRIPE = 624
TAIL_ROWS = N_NODES - NS * STRIPE

_MESH = plsc.VectorSubcoreMesh(core_axis_name="c", subcore_axis_name="s")


@functools.partial(
    pl.kernel,
    out_type=[jax.ShapeDtypeStruct((N_NODES,), jnp.float32),
              jax.ShapeDtypeStruct((N_NODES,), jnp.float32)],
    mesh=_MESH,
    scratch_types=[
        pltpu.VMEM((DEG_ROWS_PER_W, DEG_CHUNK), jnp.int32),
        pltpu.VMEM((DEG_CHUNK,), jnp.float32),
        pltpu.VMEM_SHARED((N_NODES,), jnp.float32),
        pltpu.SemaphoreType.DMA,
    ],
)
def _deg_kernel(src_hbm, zeros_hbm, ones_hbm, deg0_hbm, deg1_hbm,
                idx_v, ones_v, deg_sh, sem):
    c = lax.axis_index("c")
    s = lax.axis_index("s")
    w = c * NS + s

    @pl.when(s == 0)
    def _():
        pltpu.sync_copy(zeros_hbm, deg_sh)

    pltpu.sync_copy(src_hbm.at[pl.ds(w * DEG_ROWS_PER_W, DEG_ROWS_PER_W)],
                    idx_v)
    pltpu.sync_copy(ones_hbm, ones_v)
    plsc.subcore_barrier()

    @pl.loop(0, DEG_ROWS_PER_W)
    def _(j):
        pltpu.sync_copy(ones_v, deg_sh.at[idx_v.at[j]], add=True)

    plsc.subcore_barrier()

    @pl.when((s == 0) & (c == 0))
    def _():
        pltpu.sync_copy(deg_sh, deg0_hbm)

    @pl.when((s == 0) & (c == 1))
    def _():
        pltpu.sync_copy(deg_sh, deg1_hbm)


_TC_BLOCK = 2000


def _tc_body(x_ref, w_ref, deg0_ref, deg1_ref, h_ref):
    deg = deg0_ref[:, 0] + deg1_ref[:, 0] + 1.0
    norm = lax.rsqrt(deg)
    h = jnp.dot(x_ref[...], w_ref[...], preferred_element_type=jnp.float32)
    h = h * norm[:, None]
    h_ref[0] = h[:, :FH]
    h_ref[1] = h[:, FH:]


def _tc_matmul(x, W, deg0, deg1):
    grid = (N_NODES // _TC_BLOCK,)
    return pl.pallas_call(
        _tc_body,
        grid=grid,
        in_specs=[
            pl.BlockSpec((_TC_BLOCK, F_IN), lambda i: (i, 0)),
            pl.BlockSpec((F_IN, F_OUT), lambda i: (0, 0)),
            pl.BlockSpec((_TC_BLOCK, 1), lambda i: (i, 0)),
            pl.BlockSpec((_TC_BLOCK, 1), lambda i: (i, 0)),
        ],
        out_specs=pl.BlockSpec((NC, _TC_BLOCK, FH), lambda i: (0, i, 0)),
        out_shape=jax.ShapeDtypeStruct((NC, N_NODES, FH), jnp.float32),
    )(x, W, deg0, deg1)


@functools.partial(
    pl.kernel,
    out_type=jax.ShapeDtypeStruct((N_NODES, F_OUT), jnp.float32),
    mesh=_MESH,
    scratch_types=[
        pltpu.VMEM((AGG_ROWS_PER_W, AGG_CHUNK), jnp.int32),
        pltpu.VMEM((8, AGG_CHUNK), jnp.int32),
        pltpu.VMEM((AGG_CHUNK, FH), jnp.float32),
        pltpu.VMEM((AGG_CHUNK, FH), jnp.float32),
        pltpu.VMEM_SHARED((N_NODES, FH), jnp.float32),
        pltpu.SemaphoreType.DMA,
        pltpu.SemaphoreType.DMA,
    ],
)
def _agg_kernel(hcat_hbm, srcs_hbm, dst_hbm, zeros_hbm, out_hbm,
                sidx_v, didx_g, rows_a, rows_b, acc_sh, sem_a, sem_b):
    c = lax.axis_index("c")
    s = lax.axis_index("s")

    pltpu.sync_copy(zeros_hbm.at[pl.ds(s * STRIPE, STRIPE)],
                    acc_sh.at[pl.ds(s * STRIPE, STRIPE)])

    @pl.when(s == NS - 1)
    def _():
        pltpu.sync_copy(zeros_hbm.at[pl.ds(NS * STRIPE, TAIL_ROWS)],
                        acc_sh.at[pl.ds(NS * STRIPE, TAIL_ROWS)])

    pltpu.sync_copy(
        srcs_hbm.at[c].at[pl.ds(s * AGG_ROWS_PER_W, AGG_ROWS_PER_W)],
        sidx_v)
    plsc.subcore_barrier()

    pltpu.async_copy(hcat_hbm.at[sidx_v.at[0]], rows_a, sem_a)

    @pl.loop(0, AGG_ROWS_PER_W, step=2)
    def _(j):
        @pl.when(lax.rem(j, 8) == 0)
        def _():
            base = pl.multiple_of(s * AGG_ROWS_PER_W + j, 8)
            pltpu.sync_copy(dst_hbm.at[pl.ds(base, 8)], didx_g)

        k = lax.rem(j, 8)
        pltpu.async_copy(hcat_hbm.at[sidx_v.at[j + 1]], rows_b, sem_b)
        pltpu.make_async_copy(hcat_hbm.at[sidx_v.at[j]], rows_a, sem_a).wait()
        pltpu.sync_copy(rows_a, acc_sh.at[didx_g.at[k]], add=True)

        @pl.when(j + 2 < AGG_ROWS_PER_W)
        def _():
            pltpu.async_copy(hcat_hbm.at[sidx_v.at[j + 2]], rows_a, sem_a)

        pltpu.make_async_copy(hcat_hbm.at[sidx_v.at[j + 1]], rows_b,
                              sem_b).wait()
        pltpu.sync_copy(rows_b, acc_sh.at[didx_g.at[k + 1]], add=True)

    plsc.subcore_barrier()

    pltpu.sync_copy(
        acc_sh.at[pl.ds(s * STRIPE, STRIPE)],
        out_hbm.at[pl.ds(s * STRIPE, STRIPE), pl.ds(c * FH, FH)])

    @pl.when(s == NS - 1)
    def _():
        pltpu.sync_copy(
            acc_sh.at[pl.ds(NS * STRIPE, TAIL_ROWS)],
            out_hbm.at[pl.ds(NS * STRIPE, TAIL_ROWS), pl.ds(c * FH, FH)])


def kernel(x, edge_index, W, b):
    src = edge_index[0].astype(jnp.int32)
    dst = edge_index[1].astype(jnp.int32)

    src_deg = src.reshape(NC * NS * DEG_ROWS_PER_W, DEG_CHUNK)
    zeros_1d = jnp.zeros((N_NODES,), jnp.float32)
    ones_c = jnp.ones((DEG_CHUNK,), jnp.float32)
    deg0, deg1 = _deg_kernel(src_deg, zeros_1d, ones_c)

    h = _tc_matmul(x, W, deg0.reshape(N_NODES, 1), deg1.reshape(N_NODES, 1))
    hcat = h.reshape(NC * N_NODES, FH)

    srcr = src.reshape(NS * AGG_ROWS_PER_W, AGG_CHUNK)
    srcs = jnp.stack([srcr, srcr + N_NODES])
    dstr = dst.reshape(NS * AGG_ROWS_PER_W, AGG_CHUNK)
    zeros_2d = jnp.zeros((N_NODES, FH), jnp.float32)

    out = _agg_kernel(hcat, srcs, dstr, zeros_2d)
    return out + b


# speedup vs baseline: 8.3864x; 1.0296x over previous
import functools

import jax
import jax.numpy as jnp
from jax import lax
from jax.experimental import pallas as pl
from jax.experimental.pallas import tpu as pltpu
from jax.experimental.pallas import tpu_sc as plsc

N_NODES = 10000
N_EDGES = 160000
F_IN = 256
F_OUT = 256
FH = 128
NC = 2
NS = 16

DEG_CHUNK = 125
DEG_ROWS_PER_W = (N_EDGES // (NC * NS)) // DEG_CHUNK

AGG_CHUNK = 125
AGG_ROWS_PER_W = (N_EDGES // NS) // AGG_CHUNK

STRIPE = 624
TAIL_ROWS = N_NODES - NS * STRIPE

_MESH = plsc.VectorSubcoreMesh(core_axis_name="c", subcore_axis_name="s")


@functools.partial(
    pl.kernel,
    out_type=[jax.ShapeDtypeStruct((N_NODES,), jnp.float32),
              jax.ShapeDtypeStruct((N_NODES,), jnp.float32)],
    mesh=_MESH,
    scratch_types=[
        pltpu.VMEM((DEG_ROWS_PER_W, DEG_CHUNK), jnp.int32),
        pltpu.VMEM((DEG_CHUNK,), jnp.float32),
        pltpu.VMEM_SHARED((N_NODES,), jnp.float32),
        pltpu.SemaphoreType.DMA,
    ],
)
def _deg_kernel(src_hbm, zeros_hbm, ones_hbm, deg0_hbm, deg1_hbm,
                idx_v, ones_v, deg_sh, sem):
    c = lax.axis_index("c")
    s = lax.axis_index("s")
    w = c * NS + s

    @pl.when(s == 0)
    def _():
        pltpu.sync_copy(zeros_hbm, deg_sh)

    pltpu.sync_copy(src_hbm.at[pl.ds(w * DEG_ROWS_PER_W, DEG_ROWS_PER_W)],
                    idx_v)
    pltpu.sync_copy(ones_hbm, ones_v)
    plsc.subcore_barrier()

    @pl.loop(0, DEG_ROWS_PER_W)
    def _(j):
        pltpu.sync_copy(ones_v, deg_sh.at[idx_v.at[j]], add=True)

    plsc.subcore_barrier()

    @pl.when((s == 0) & (c == 0))
    def _():
        pltpu.sync_copy(deg_sh, deg0_hbm)

    @pl.when((s == 0) & (c == 1))
    def _():
        pltpu.sync_copy(deg_sh, deg1_hbm)


_TC_BLOCK = 2000


def _tc_body(x_ref, w_ref, deg0_ref, deg1_ref, h_ref):
    deg = deg0_ref[:, 0] + deg1_ref[:, 0] + 1.0
    norm = lax.rsqrt(deg)
    h = jnp.dot(x_ref[...], w_ref[...], preferred_element_type=jnp.float32)
    h = h * norm[:, None]
    h_ref[0] = h[:, :FH]
    h_ref[1] = h[:, FH:]


def _tc_matmul(x, W, deg0, deg1):
    grid = (N_NODES // _TC_BLOCK,)
    return pl.pallas_call(
        _tc_body,
        grid=grid,
        in_specs=[
            pl.BlockSpec((_TC_BLOCK, F_IN), lambda i: (i, 0)),
            pl.BlockSpec((F_IN, F_OUT), lambda i: (0, 0)),
            pl.BlockSpec((_TC_BLOCK, 1), lambda i: (i, 0)),
            pl.BlockSpec((_TC_BLOCK, 1), lambda i: (i, 0)),
        ],
        out_specs=pl.BlockSpec((NC, _TC_BLOCK, FH), lambda i: (0, i, 0)),
        out_shape=jax.ShapeDtypeStruct((NC, N_NODES, FH), jnp.float32),
    )(x, W, deg0, deg1)


@functools.partial(
    pl.kernel,
    out_type=jax.ShapeDtypeStruct((N_NODES, F_OUT), jnp.float32),
    mesh=_MESH,
    scratch_types=[
        pltpu.VMEM((AGG_ROWS_PER_W, AGG_CHUNK), jnp.int32),
        pltpu.VMEM((8, AGG_CHUNK), jnp.int32),
        pltpu.VMEM((AGG_CHUNK, FH), jnp.float32),
        pltpu.VMEM((AGG_CHUNK, FH), jnp.float32),
        pltpu.VMEM_SHARED((N_NODES, FH), jnp.float32),
        pltpu.SemaphoreType.DMA,
        pltpu.SemaphoreType.DMA,
    ],
)
def _agg_kernel(h_hbm, src_hbm, dst_hbm, binit_hbm, out_hbm,
                sidx_v, didx_g, rows_a, rows_b, acc_sh, sem_a, sem_b):
    c = lax.axis_index("c")
    s = lax.axis_index("s")

    pltpu.sync_copy(binit_hbm.at[c].at[pl.ds(s * STRIPE, STRIPE)],
                    acc_sh.at[pl.ds(s * STRIPE, STRIPE)])

    @pl.when(s == NS - 1)
    def _():
        pltpu.sync_copy(binit_hbm.at[c].at[pl.ds(NS * STRIPE, TAIL_ROWS)],
                        acc_sh.at[pl.ds(NS * STRIPE, TAIL_ROWS)])

    pltpu.sync_copy(
        src_hbm.at[pl.ds(s * AGG_ROWS_PER_W, AGG_ROWS_PER_W)],
        sidx_v)
    plsc.subcore_barrier()

    tab = h_hbm.at[c]

    pltpu.async_copy(tab.at[sidx_v.at[0]], rows_a, sem_a)

    @pl.loop(0, AGG_ROWS_PER_W, step=2)
    def _(j):
        @pl.when(lax.rem(j, 8) == 0)
        def _():
            base = pl.multiple_of(s * AGG_ROWS_PER_W + j, 8)
            pltpu.sync_copy(dst_hbm.at[pl.ds(base, 8)], didx_g)

        k = lax.rem(j, 8)
        pltpu.async_copy(tab.at[sidx_v.at[j + 1]], rows_b, sem_b)
        pltpu.make_async_copy(tab.at[sidx_v.at[j]], rows_a, sem_a).wait()
        pltpu.sync_copy(rows_a, acc_sh.at[didx_g.at[k]], add=True)

        @pl.when(j + 2 < AGG_ROWS_PER_W)
        def _():
            pltpu.async_copy(tab.at[sidx_v.at[j + 2]], rows_a, sem_a)

        pltpu.make_async_copy(tab.at[sidx_v.at[j + 1]], rows_b,
                              sem_b).wait()
        pltpu.sync_copy(rows_b, acc_sh.at[didx_g.at[k + 1]], add=True)

    plsc.subcore_barrier()

    pltpu.sync_copy(
        acc_sh.at[pl.ds(s * STRIPE, STRIPE)],
        out_hbm.at[pl.ds(s * STRIPE, STRIPE), pl.ds(c * FH, FH)])

    @pl.when(s == NS - 1)
    def _():
        pltpu.sync_copy(
            acc_sh.at[pl.ds(NS * STRIPE, TAIL_ROWS)],
            out_hbm.at[pl.ds(NS * STRIPE, TAIL_ROWS), pl.ds(c * FH, FH)])


def kernel(x, edge_index, W, b):
    src = edge_index[0].astype(jnp.int32)
    dst = edge_index[1].astype(jnp.int32)

    src_deg = src.reshape(NC * NS * DEG_ROWS_PER_W, DEG_CHUNK)
    zeros_1d = jnp.zeros((N_NODES,), jnp.float32)
    ones_c = jnp.ones((DEG_CHUNK,), jnp.float32)
    deg0, deg1 = _deg_kernel(src_deg, zeros_1d, ones_c)

    h = _tc_matmul(x, W, deg0.reshape(N_NODES, 1), deg1.reshape(N_NODES, 1))

    srcr = src.reshape(NS * AGG_ROWS_PER_W, AGG_CHUNK)
    dstr = dst.reshape(NS * AGG_ROWS_PER_W, AGG_CHUNK)
    binit = jnp.broadcast_to(b.reshape(NC, 1, FH), (NC, N_NODES, FH))

    return _agg_kernel(h, srcr, dstr, binit)
